# Initial kernel scaffold; baseline (speedup 1.0000x reference)
import jax
import jax.numpy as jnp
from jax.experimental import pallas as pl


def kernel(x, edge_index, W1, b1, W2, b2):
    raise NotImplementedError("write your pallas kernel here")


# speedup vs baseline: 22.7346x; 22.7346x over previous
import functools

import jax
import jax.numpy as jnp
from jax import lax
from jax.experimental import pallas as pl
from jax.experimental.pallas import tpu as pltpu
from jax.experimental.pallas import tpu_sc as plsc

N = 10000
E = 320000
D_IN = 128
D_H = 64
D_OUT = 32

NC = 2
NS = 16
CH = 128
K = 80
TOT = NC * NS * K * CH
N_PAD = 10240
RPT = N_PAD // NS

_mesh = plsc.VectorSubcoreMesh(core_axis_name="c", subcore_axis_name="s")


def _sc_degree(dsti, zeros1):
    @functools.partial(
        pl.kernel,
        mesh=_mesh,
        compiler_params=pltpu.CompilerParams(use_tc_tiling_on_sc=False),
        out_type=jax.ShapeDtypeStruct((NC, N_PAD), jnp.float32),
        scratch_types=[
            pltpu.VMEM((K, CH), jnp.int32),
            pltpu.VMEM((CH,), jnp.float32),
            pltpu.VMEM_SHARED((N_PAD,), jnp.float32),
        ],
    )
    def k(dst_h, z_h, out, dst_v, ones_v, dacc):
        cid = lax.axis_index("c")
        sid = lax.axis_index("s")
        pltpu.sync_copy(dst_h.at[cid, sid], dst_v)
        for t in range(CH // 16):
            ones_v[pl.ds(16 * t, 16)] = jnp.full((16,), 1.0, jnp.float32)
        r0 = sid * RPT
        pltpu.sync_copy(z_h.at[pl.ds(r0, RPT)], dacc.at[pl.ds(r0, RPT)])
        plsc.subcore_barrier()

        def body(j, carry):
            pltpu.sync_copy(ones_v, dacc.at[dst_v.at[j]], add=True)
            return carry

        lax.fori_loop(0, K, body, 0)
        plsc.subcore_barrier()
        pltpu.sync_copy(dacc.at[pl.ds(r0, RPT)], out.at[cid, pl.ds(r0, RPT)])

    return k(dsti, zeros1)


def _sc_edge_pass(y, srci, dsti, zeros2, d):
    @functools.partial(
        pl.kernel,
        mesh=_mesh,
        compiler_params=pltpu.CompilerParams(use_tc_tiling_on_sc=False),
        out_type=jax.ShapeDtypeStruct((NC, N_PAD, d), jnp.float32),
        scratch_types=[
            pltpu.VMEM((K, CH), jnp.int32),
            pltpu.VMEM((K, CH), jnp.int32),
            pltpu.VMEM((CH, d), jnp.float32),
            pltpu.VMEM((CH, d), jnp.float32),
            pltpu.VMEM_SHARED((N_PAD, d), jnp.float32),
            pltpu.SemaphoreType.DMA,
            pltpu.SemaphoreType.DMA,
        ],
    )
    def k(y_h, src_h, dst_h, z_h, out, src_v, dst_v, rows0, rows1, acc,
          sem0, sem1):
        cid = lax.axis_index("c")
        sid = lax.axis_index("s")
        pltpu.sync_copy(src_h.at[cid, sid], src_v)
        pltpu.sync_copy(dst_h.at[cid, sid], dst_v)
        r0 = sid * RPT
        pltpu.sync_copy(z_h.at[pl.ds(r0, RPT)], acc.at[pl.ds(r0, RPT)])
        plsc.subcore_barrier()

        pltpu.make_async_copy(y_h.at[src_v.at[0]], rows0, sem0).start()
        pltpu.make_async_copy(y_h.at[src_v.at[1]], rows1, sem1).start()

        def body(g, carry):
            j0 = 2 * g
            pltpu.make_async_copy(y_h.at[src_v.at[j0]], rows0, sem0).wait()
            pltpu.sync_copy(rows0, acc.at[dst_v.at[j0]], add=True)
            pltpu.make_async_copy(y_h.at[src_v.at[j0 + 2]], rows0, sem0).start()
            j1 = j0 + 1
            pltpu.make_async_copy(y_h.at[src_v.at[j1]], rows1, sem1).wait()
            pltpu.sync_copy(rows1, acc.at[dst_v.at[j1]], add=True)
            pltpu.make_async_copy(y_h.at[src_v.at[j1 + 2]], rows1, sem1).start()
            return carry

        lax.fori_loop(0, K // 2 - 1, body, 0)
        pltpu.make_async_copy(y_h.at[src_v.at[K - 2]], rows0, sem0).wait()
        pltpu.sync_copy(rows0, acc.at[dst_v.at[K - 2]], add=True)
        pltpu.make_async_copy(y_h.at[src_v.at[K - 1]], rows1, sem1).wait()
        pltpu.sync_copy(rows1, acc.at[dst_v.at[K - 1]], add=True)

        plsc.subcore_barrier()
        pltpu.sync_copy(acc.at[pl.ds(r0, RPT)], out.at[cid, pl.ds(r0, RPT)])

    return k(y, srci, dsti, zeros2)


def _tc_scale_mm(deg2, x_pad, W1):
    def body(deg_ref, x_ref, w_ref, d_ref, y_ref):
        deg = deg_ref[0] + deg_ref[1] + 1.0
        dcol = lax.rsqrt(deg)
        d_ref[...] = dcol
        xw = jnp.dot(x_ref[...], w_ref[...],
                     preferred_element_type=jnp.float32)
        y_ref[...] = xw * dcol

    return pl.pallas_call(
        body,
        out_shape=[
            jax.ShapeDtypeStruct((N_PAD, 1), jnp.float32),
            jax.ShapeDtypeStruct((N_PAD, D_H), jnp.float32),
        ],
    )(deg2, x_pad, W1)


def _tc_mid(acc1, y1, d, b1, W2):
    def body(acc_ref, y_ref, d_ref, b_ref, w_ref, y2_ref):
        s = acc_ref[0] + acc_ref[1]
        dcol = d_ref[...]
        h = jnp.maximum((s + y_ref[...]) * dcol + b_ref[...], 0.0)
        y2_ref[...] = jnp.dot(h, w_ref[...],
                              preferred_element_type=jnp.float32) * dcol

    return pl.pallas_call(
        body,
        out_shape=jax.ShapeDtypeStruct((N_PAD, D_OUT), jnp.float32),
    )(acc1, y1, d, b1.reshape(1, D_H), W2)


def _tc_final(acc2, y2, d, b2):
    def body(acc_ref, y_ref, d_ref, b_ref, o_ref):
        s = acc_ref[0] + acc_ref[1]
        o_ref[...] = jnp.maximum(
            (s + y_ref[...]) * d_ref[...] + b_ref[...], 0.0)

    return pl.pallas_call(
        body,
        out_shape=jax.ShapeDtypeStruct((N_PAD, D_OUT), jnp.float32),
    )(acc2, y2, d, b2.reshape(1, D_OUT))


def kernel(x, edge_index, W1, b1, W2, b2):
    src = edge_index[0]
    dst = edge_index[1]
    pad = TOT - E
    src_p = jnp.concatenate([src, jnp.zeros((pad,), jnp.int32)])
    dst_p = jnp.concatenate([dst, jnp.full((pad,), N, jnp.int32)])
    srci = src_p.reshape(NC, NS, K, CH)
    dsti = dst_p.reshape(NC, NS, K, CH)

    x_pad = jnp.pad(x, ((0, N_PAD - N), (0, 0)))
    z1 = jnp.zeros((N_PAD,), jnp.float32)
    zH = jnp.zeros((N_PAD, D_H), jnp.float32)
    zO = jnp.zeros((N_PAD, D_OUT), jnp.float32)

    deg2 = _sc_degree(dsti, z1)
    d, y1 = _tc_scale_mm(deg2.reshape(NC, N_PAD, 1), x_pad, W1)
    acc1 = _sc_edge_pass(y1, srci, dsti, zH, D_H)
    y2 = _tc_mid(acc1, y1, d, b1, W2)
    acc2 = _sc_edge_pass(y2, srci, dsti, zO, D_OUT)
    out = _tc_final(acc2, y2, d, b2)
    return out[:N]
